# baseline (speedup 1.0000x reference)
import jax
import jax.numpy as jnp
from jax import lax
from jax.experimental import pallas as pl
from jax.experimental.pallas import tpu as pltpu

_CHUNK = 8192
_SUB = 4
_W = _CHUNK // _SUB


def _norm_rows(x):
    return x / (jnp.sqrt(jnp.sum(x * x, axis=-1, keepdims=True)) + 1e-8)


def _norm_cols(xt):
    return xt / (jnp.sqrt(jnp.sum(xt * xt, axis=0, keepdims=True)) + 1e-8)


def _argmax_rows_first(s, n):
    m = jnp.max(s, axis=-1, keepdims=True)
    iota = lax.broadcasted_iota(jnp.int32, s.shape, 1)
    return jnp.min(jnp.where(s == m, iota, n), axis=-1, keepdims=True)


def _argmax_cols_first(s, n):
    m = jnp.max(s, axis=0, keepdims=True)
    iota = lax.broadcasted_iota(jnp.int32, s.shape, 0)
    return jnp.min(jnp.where(s == m, iota, n), axis=0, keepdims=True)


def _cluster_kernel(xt_ref, cc_ref, sc_ref, y_ref, mean_ref,
                    oh_scr, sums_scr, counts_scr):
    k = pl.program_id(1)
    xt = xt_ref[...]
    cc = cc_ref[...]
    sc = sc_ref[...]
    C = cc.shape[0]
    n = xt.shape[1]

    xnt = _norm_cols(xt)
    ccn = _norm_rows(cc)
    sim = lax.dot_general(ccn, xnt, (((1,), (0,)), ((), ())))
    local = _argmax_cols_first(sim, C)

    iota_c = lax.broadcasted_iota(jnp.int32, (C, n), 0)
    onehot_f = (iota_c == local).astype(jnp.float32)
    onehot = onehot_f.astype(jnp.bfloat16)
    oh_scr[:, pl.ds(k * _W, _W)] = onehot

    x_hi = xt.astype(jnp.bfloat16)
    r1 = xt - x_hi.astype(jnp.float32)
    x_mid = r1.astype(jnp.bfloat16)
    x_lo = (r1 - x_mid.astype(jnp.float32)).astype(jnp.bfloat16)
    dims = (((1,), (1,)), ((), ()))
    psums = (lax.dot_general(onehot, x_hi, dims,
                             preferred_element_type=jnp.float32)
             + lax.dot_general(onehot, x_mid, dims,
                               preferred_element_type=jnp.float32)
             + lax.dot_general(onehot, x_lo, dims,
                               preferred_element_type=jnp.float32))
    ones = jnp.ones((1, n), jnp.float32)
    pcounts = lax.dot_general(onehot_f, ones, (((1,), (1,)), ((), ())))

    @pl.when(k == 0)
    def _init():
        sums_scr[...] = psums
        counts_scr[...] = pcounts

    @pl.when(k > 0)
    def _acc():
        sums_scr[...] += psums
        counts_scr[...] += pcounts

    @pl.when(k == _SUB - 1)
    def _finalize():
        sums = sums_scr[...]
        counts = counts_scr[...]
        mean = sums / jnp.maximum(counts, 1.0)

        meann = _norm_rows(mean)
        scn = _norm_rows(sc)
        spk = lax.dot_general(meann, scn, (((1,), (1,)), ((), ())))
        agg = _argmax_rows_first(spk, sc.shape[0])

        agg_row = jnp.transpose(agg, (1, 0)).astype(jnp.bfloat16)
        y = lax.dot_general(agg_row, oh_scr[...],
                            (((1,), (0,)), ((), ())),
                            preferred_element_type=jnp.float32)
        y_ref[...] = y.astype(jnp.int32).reshape(1, 1, _CHUNK)
        mean_ref[...] = mean


def kernel(embeddings, chunk_centroids, speaker_centroids,
           embeddings_per_chunk, chunk_cluster_count, max_num_speakers):
    N, D = embeddings.shape
    C = chunk_centroids.shape[0]
    S = speaker_centroids.shape[0]
    n_chunks = N // _CHUNK
    num_seg = n_chunks * C

    emb_t = embeddings.T

    y2, mean = pl.pallas_call(
        _cluster_kernel,
        grid=(n_chunks, _SUB),
        in_specs=[
            pl.BlockSpec((D, _W), lambda i, k: (0, i * _SUB + k)),
            pl.BlockSpec((C, D), lambda i, k: (0, 0)),
            pl.BlockSpec((S, D), lambda i, k: (0, 0)),
        ],
        out_specs=[
            pl.BlockSpec((1, 1, _CHUNK), lambda i, k: (i, 0, 0)),
            pl.BlockSpec((C, D), lambda i, k: (i, 0)),
        ],
        out_shape=[
            jax.ShapeDtypeStruct((n_chunks, 1, _CHUNK), jnp.int32),
            jax.ShapeDtypeStruct((num_seg, D), jnp.float32),
        ],
        scratch_shapes=[
            pltpu.VMEM((C, _CHUNK), jnp.bfloat16),
            pltpu.VMEM((C, D), jnp.float32),
            pltpu.VMEM((C, 1), jnp.float32),
        ],
    )(emb_t, chunk_centroids, speaker_centroids)
    return y2.reshape(N), mean


# speedup vs baseline: 1.1650x; 1.1650x over previous
import jax
import jax.numpy as jnp
from jax import lax
from jax.experimental import pallas as pl
from jax.experimental.pallas import tpu as pltpu

_CHUNK = 8192
_SUB = 2
_W = _CHUNK // _SUB


def _norm_rows(x):
    return x / (jnp.sqrt(jnp.sum(x * x, axis=-1, keepdims=True)) + 1e-8)


def _norm_cols(xt):
    return xt / (jnp.sqrt(jnp.sum(xt * xt, axis=0, keepdims=True)) + 1e-8)


def _argmax_rows_first(s, n):
    m = jnp.max(s, axis=-1, keepdims=True)
    iota = lax.broadcasted_iota(jnp.int32, s.shape, 1)
    return jnp.min(jnp.where(s == m, iota, n), axis=-1, keepdims=True)


def _argmax_cols_first(s, n):
    m = jnp.max(s, axis=0, keepdims=True)
    iota = lax.broadcasted_iota(jnp.int32, s.shape, 0)
    return jnp.min(jnp.where(s == m, iota, n), axis=0, keepdims=True)


def _cluster_kernel(xt_ref, cc_ref, sc_ref, y_ref, mean_ref,
                    oh_scr, sums_scr, counts_scr):
    k = pl.program_id(1)
    xt = xt_ref[...]
    cc = cc_ref[...]
    sc = sc_ref[...]
    C = cc.shape[0]
    n = xt.shape[1]

    xnt = _norm_cols(xt)
    ccn = _norm_rows(cc)
    sim = lax.dot_general(ccn, xnt, (((1,), (0,)), ((), ())))
    local = _argmax_cols_first(sim, C)

    iota_c = lax.broadcasted_iota(jnp.int32, (C, n), 0)
    onehot_f = (iota_c == local).astype(jnp.float32)
    oh_scr[:, pl.ds(k * _W, _W)] = onehot_f

    h = xt.astype(jnp.bfloat16).astype(jnp.float32)
    r1 = xt - h
    m = r1.astype(jnp.bfloat16).astype(jnp.float32)
    r2 = r1 - m
    dims = (((1,), (1,)), ((), ()))
    psums = (lax.dot_general(onehot_f, h, dims)
             + lax.dot_general(onehot_f, m, dims)
             + lax.dot_general(onehot_f, r2, dims))
    ones = jnp.ones((1, n), jnp.float32)
    pcounts = lax.dot_general(onehot_f, ones, (((1,), (1,)), ((), ())))

    @pl.when(k == 0)
    def _init():
        sums_scr[...] = psums
        counts_scr[...] = pcounts

    @pl.when(k > 0)
    def _acc():
        sums_scr[...] += psums
        counts_scr[...] += pcounts

    @pl.when(k == _SUB - 1)
    def _finalize():
        sums = sums_scr[...]
        counts = counts_scr[...]
        mean = sums / jnp.maximum(counts, 1.0)

        meann = _norm_rows(mean)
        scn = _norm_rows(sc)
        spk = lax.dot_general(meann, scn, (((1,), (1,)), ((), ())))
        agg = _argmax_rows_first(spk, sc.shape[0])

        agg_row = jnp.transpose(agg, (1, 0)).astype(jnp.float32)
        y = lax.dot_general(agg_row, oh_scr[...],
                            (((1,), (0,)), ((), ())))
        y_ref[...] = y.astype(jnp.int32).reshape(1, 1, _CHUNK)
        mean_ref[...] = mean


def kernel(embeddings, chunk_centroids, speaker_centroids,
           embeddings_per_chunk, chunk_cluster_count, max_num_speakers):
    N, D = embeddings.shape
    C = chunk_centroids.shape[0]
    S = speaker_centroids.shape[0]
    n_chunks = N // _CHUNK
    num_seg = n_chunks * C

    emb_t = embeddings.T

    y2, mean = pl.pallas_call(
        _cluster_kernel,
        grid=(n_chunks, _SUB),
        in_specs=[
            pl.BlockSpec((D, _W), lambda i, k: (0, i * _SUB + k)),
            pl.BlockSpec((C, D), lambda i, k: (0, 0)),
            pl.BlockSpec((S, D), lambda i, k: (0, 0)),
        ],
        out_specs=[
            pl.BlockSpec((1, 1, _CHUNK), lambda i, k: (i, 0, 0)),
            pl.BlockSpec((C, D), lambda i, k: (i, 0)),
        ],
        out_shape=[
            jax.ShapeDtypeStruct((n_chunks, 1, _CHUNK), jnp.int32),
            jax.ShapeDtypeStruct((num_seg, D), jnp.float32),
        ],
        scratch_shapes=[
            pltpu.VMEM((C, _CHUNK), jnp.float32),
            pltpu.VMEM((C, D), jnp.float32),
            pltpu.VMEM((C, 1), jnp.float32),
        ],
    )(emb_t, chunk_centroids, speaker_centroids)
    return y2.reshape(N), mean


# speedup vs baseline: 1.1651x; 1.0001x over previous
import jax
import jax.numpy as jnp
from jax import lax
from jax.experimental import pallas as pl
from jax.experimental.pallas import tpu as pltpu

_CHUNK = 8192
_SUB = 2
_W = _CHUNK // _SUB


def _norm_rows(x):
    return x / (jnp.sqrt(jnp.sum(x * x, axis=-1, keepdims=True)) + 1e-8)


def _norm_cols(xt):
    return xt / (jnp.sqrt(jnp.sum(xt * xt, axis=0, keepdims=True)) + 1e-8)


def _argmax_rows_first(s, n):
    m = jnp.max(s, axis=-1, keepdims=True)
    iota = lax.broadcasted_iota(jnp.int32, s.shape, 1)
    return jnp.min(jnp.where(s == m, iota, n), axis=-1, keepdims=True)


def _argmax_cols_first(s, n):
    m = jnp.max(s, axis=0, keepdims=True)
    iota = lax.broadcasted_iota(jnp.int32, s.shape, 0)
    return jnp.min(jnp.where(s == m, iota, n), axis=0, keepdims=True)


def _cluster_kernel(xt_ref, cc_ref, sc_ref, y_ref, mean_ref,
                    oh_scr, sums_scr):
    k = pl.program_id(1)
    xt = xt_ref[...]
    cc = cc_ref[...]
    sc = sc_ref[...]
    C = cc.shape[0]
    n = xt.shape[1]

    xnt = _norm_cols(xt)
    ccn = _norm_rows(cc)
    sim = lax.dot_general(ccn, xnt, (((1,), (0,)), ((), ())))
    local = _argmax_cols_first(sim, C)

    iota_c = lax.broadcasted_iota(jnp.int32, (C, n), 0)
    onehot_f = (iota_c == local).astype(jnp.float32)
    oh_scr[:, pl.ds(k * _W, _W)] = onehot_f

    h = xt.astype(jnp.bfloat16).astype(jnp.float32)
    r1 = xt - h
    m = r1.astype(jnp.bfloat16).astype(jnp.float32)
    r2 = r1 - m
    dims = (((1,), (1,)), ((), ()))
    psums = (lax.dot_general(onehot_f, h, dims)
             + lax.dot_general(onehot_f, m, dims)
             + lax.dot_general(onehot_f, r2, dims))

    @pl.when(k == 0)
    def _init():
        sums_scr[...] = psums

    @pl.when(k > 0)
    def _acc():
        sums_scr[...] += psums

    @pl.when(k == _SUB - 1)
    def _finalize():
        sums = sums_scr[...]
        ones = jnp.ones((1, _CHUNK), jnp.float32)
        counts = lax.dot_general(oh_scr[...], ones,
                                 (((1,), (1,)), ((), ())))
        mean = sums / jnp.maximum(counts, 1.0)

        meann = _norm_rows(mean)
        scn = _norm_rows(sc)
        spk = lax.dot_general(meann, scn, (((1,), (1,)), ((), ())))
        agg = _argmax_rows_first(spk, sc.shape[0])

        agg_row = jnp.transpose(agg, (1, 0)).astype(jnp.float32)
        y = lax.dot_general(agg_row, oh_scr[...],
                            (((1,), (0,)), ((), ())))
        y_ref[...] = y.astype(jnp.int32).reshape(1, 1, _CHUNK)
        mean_ref[...] = mean


def kernel(embeddings, chunk_centroids, speaker_centroids,
           embeddings_per_chunk, chunk_cluster_count, max_num_speakers):
    N, D = embeddings.shape
    C = chunk_centroids.shape[0]
    S = speaker_centroids.shape[0]
    n_chunks = N // _CHUNK
    num_seg = n_chunks * C

    emb_t = embeddings.T

    y2, mean = pl.pallas_call(
        _cluster_kernel,
        grid=(n_chunks, _SUB),
        in_specs=[
            pl.BlockSpec((D, _W), lambda i, k: (0, i * _SUB + k)),
            pl.BlockSpec((C, D), lambda i, k: (0, 0)),
            pl.BlockSpec((S, D), lambda i, k: (0, 0)),
        ],
        out_specs=[
            pl.BlockSpec((1, 1, _CHUNK), lambda i, k: (i, 0, 0)),
            pl.BlockSpec((C, D), lambda i, k: (i, 0)),
        ],
        out_shape=[
            jax.ShapeDtypeStruct((n_chunks, 1, _CHUNK), jnp.int32),
            jax.ShapeDtypeStruct((num_seg, D), jnp.float32),
        ],
        scratch_shapes=[
            pltpu.VMEM((C, _CHUNK), jnp.float32),
            pltpu.VMEM((C, D), jnp.float32),
        ],
    )(emb_t, chunk_centroids, speaker_centroids)
    return y2.reshape(N), mean


# speedup vs baseline: 1.1690x; 1.0033x over previous
import jax
import jax.numpy as jnp
from jax import lax
from jax.experimental import pallas as pl
from jax.experimental.pallas import tpu as pltpu

_CHUNK = 8192
_SUB = 2
_W = _CHUNK // _SUB


def _norm_rows(x):
    return x / (jnp.sqrt(jnp.sum(x * x, axis=-1, keepdims=True)) + 1e-8)


def _norm_cols(xt):
    return xt / (jnp.sqrt(jnp.sum(xt * xt, axis=0, keepdims=True)) + 1e-8)


def _argmax_rows_first(s, n):
    m = jnp.max(s, axis=-1, keepdims=True)
    iota = lax.broadcasted_iota(jnp.int32, s.shape, 1)
    return jnp.min(jnp.where(s == m, iota, n), axis=-1, keepdims=True)


def _argmax_cols_first(s, n):
    m = jnp.max(s, axis=0, keepdims=True)
    iota = lax.broadcasted_iota(jnp.int32, s.shape, 0)
    return jnp.min(jnp.where(s == m, iota, n), axis=0, keepdims=True)


def _cluster_kernel(xt_ref, cc_ref, sc_ref, y_ref, mean_ref,
                    oh_scr, sums_scr):
    k = pl.program_id(1)
    xt = xt_ref[...]
    cc = cc_ref[...]
    sc = sc_ref[...]
    C = cc.shape[0]
    n = xt.shape[1]

    xnt = _norm_cols(xt)
    ccn = _norm_rows(cc)
    sim = lax.dot_general(ccn, xnt, (((1,), (0,)), ((), ())))
    local = _argmax_cols_first(sim, C)

    iota_c = lax.broadcasted_iota(jnp.int32, (C, n), 0)
    onehot_f = (iota_c == local).astype(jnp.float32)
    oh_scr[:, pl.ds(k * _W, _W)] = onehot_f

    h = xt.astype(jnp.bfloat16).astype(jnp.float32)
    r1 = xt - h
    m = r1.astype(jnp.bfloat16).astype(jnp.float32)
    r2 = r1 - m
    dims = (((1,), (1,)), ((), ()))
    psums = (lax.dot_general(onehot_f, h, dims)
             + lax.dot_general(onehot_f, m, dims)
             + lax.dot_general(onehot_f, r2, dims))

    @pl.when(k == 0)
    def _init():
        sums_scr[...] = psums

    @pl.when(k > 0)
    def _acc():
        sums_scr[...] += psums

    @pl.when(k == _SUB - 1)
    def _finalize():
        sums = sums_scr[...]
        ones = jnp.ones((1, _CHUNK), jnp.float32)
        counts = lax.dot_general(oh_scr[...], ones,
                                 (((1,), (1,)), ((), ())))
        mean = sums / jnp.maximum(counts, 1.0)

        meann = _norm_rows(mean)
        scn = _norm_rows(sc)
        spk = lax.dot_general(meann, scn, (((1,), (1,)), ((), ())))
        agg = _argmax_rows_first(spk, sc.shape[0])

        agg_row = jnp.transpose(agg, (1, 0)).astype(jnp.float32)
        y = lax.dot_general(agg_row, oh_scr[...],
                            (((1,), (0,)), ((), ())))
        y_ref[...] = y.astype(jnp.int32).reshape(1, 1, _CHUNK)
        mean_ref[...] = mean


def kernel(embeddings, chunk_centroids, speaker_centroids,
           embeddings_per_chunk, chunk_cluster_count, max_num_speakers):
    N, D = embeddings.shape
    C = chunk_centroids.shape[0]
    S = speaker_centroids.shape[0]
    n_chunks = N // _CHUNK
    num_seg = n_chunks * C

    emb_t = embeddings.T

    y2, mean = pl.pallas_call(
        _cluster_kernel,
        grid=(n_chunks, _SUB),
        in_specs=[
            pl.BlockSpec((D, _W), lambda i, k: (0, i * _SUB + k)),
            pl.BlockSpec((C, D), lambda i, k: (0, 0)),
            pl.BlockSpec((S, D), lambda i, k: (0, 0)),
        ],
        out_specs=[
            pl.BlockSpec((1, 1, _CHUNK), lambda i, k: (i, 0, 0)),
            pl.BlockSpec((C, D), lambda i, k: (i, 0)),
        ],
        out_shape=[
            jax.ShapeDtypeStruct((n_chunks, 1, _CHUNK), jnp.int32),
            jax.ShapeDtypeStruct((num_seg, D), jnp.float32),
        ],
        scratch_shapes=[
            pltpu.VMEM((C, _CHUNK), jnp.float32),
            pltpu.VMEM((C, D), jnp.float32),
        ],
        compiler_params=pltpu.CompilerParams(
            dimension_semantics=("parallel", "arbitrary")),
    )(emb_t, chunk_centroids, speaker_centroids)
    return y2.reshape(N), mean
